# baseline (speedup 1.0000x reference)
import functools

import jax
import jax.numpy as jnp
from jax import lax
from jax.experimental import pallas as pl
from jax.experimental.pallas import tpu as pltpu
from jax.experimental.pallas import tpu_sc as plsc

_NC = 2
_NS = 16
_CH = 128
_CR = 2
_DP = 128


def _sc_gather(table, idx2d, dists_flat, e_pad, per_w):
  assert per_w % 2 == 0
  mesh = plsc.VectorSubcoreMesh(core_axis_name="c", subcore_axis_name="s")

  @functools.partial(
      pl.kernel,
      mesh=mesh,
      compiler_params=pltpu.CompilerParams(needs_layout_passes=False),
      out_type=jax.ShapeDtypeStruct((e_pad, _DP), jnp.float32),
      scratch_types=[
          pltpu.VMEM((2, _CR, _CH), jnp.int32),
          pltpu.VMEM((2, _CR * _CH), jnp.float32),
          pltpu.VMEM((2, _CR * _CH, _DP), jnp.float32),
          pltpu.SemaphoreType.DMA, pltpu.SemaphoreType.DMA,
          pltpu.SemaphoreType.DMA, pltpu.SemaphoreType.DMA,
          pltpu.SemaphoreType.DMA, pltpu.SemaphoreType.DMA,
          pltpu.SemaphoreType.DMA, pltpu.SemaphoreType.DMA,
      ],
  )
  def gather_kernel(table_hbm, idx_hbm, dist_hbm, out_hbm, idx_v, dist_v,
                    rows_v, si0, si1, sd0, sd1, so0, so1, sg0, sg1):
    wid = lax.axis_index("s") * _NC + lax.axis_index("c")
    base = wid * per_w
    lanes = lax.iota(jnp.int32, 16)
    col3 = jnp.full((16,), 3, jnp.int32)
    si = (si0, si1)
    sd = (sd0, sd1)
    so = (so0, so1)
    sg = (sg0, sg1)

    ch = _CR * _CH

    def in_idx(row, b):
      return pltpu.make_async_copy(idx_hbm.at[pl.ds(row * _CR, _CR)],
                                   idx_v.at[b], si[b])

    def in_dist(row, b):
      return pltpu.make_async_copy(dist_hbm.at[pl.ds(row * ch, ch)],
                                   dist_v.at[b], sd[b])

    def gathers(b):
      return [
          pltpu.make_async_copy(table_hbm.at[idx_v.at[b, r]],
                                rows_v.at[b, pl.ds(r * _CH, _CH)], sg[b])
          for r in range(_CR)
      ]

    def out_copy(row, b):
      return pltpu.make_async_copy(rows_v.at[b],
                                   out_hbm.at[pl.ds(row * ch, ch)], so[b])

    in_idx(base, 0).start()
    in_dist(base, 0).start()
    in_idx(base + 1, 1).start()
    in_dist(base + 1, 1).start()
    in_idx(base, 0).wait()
    for cp in gathers(0):
      cp.start()

    def body(jj, carry):
      for b in (0, 1):
        j = jj * 2 + b
        nb = 1 - b
        row = base + j

        @pl.when(j + 1 < per_w)
        def _fire_next_gather():
          in_idx(row + 1, nb).wait()

          @pl.when(j >= 1)
          def _wait_old_out():
            out_copy(row - 1, nb).wait()

          for cp in gathers(nb):
            cp.start()

        for cp in gathers(b):
          cp.wait()
        in_dist(row, b).wait()
        for grp in range(ch // 16):
          dvec = dist_v[b, pl.ds(grp * 16, 16)]
          plsc.store_scatter(rows_v.at[b], [grp * 16 + lanes, col3], dvec)
        out_copy(row, b).start()

        @pl.when(j + 2 < per_w)
        def _start_next_in():
          in_idx(row + 2, b).start()
          in_dist(row + 2, b).start()
      return carry

    lax.fori_loop(0, per_w // 2, body, 0)
    out_copy(base + per_w - 2, 0).wait()
    out_copy(base + per_w - 1, 1).wait()

  return gather_kernel(table, idx2d, dists_flat)


def _tc_compute(g, xyzn, pts, w1p, w1a, b1, w2p, b2p, w3p, w3a, b3, w4, w5,
                b45, n, k, c, blk):
  grid = (n // blk,)
  rows = blk * k

  def body(g_ref, xyzn_ref, pts_ref, w1p_r, w1a_r, b1_r, w2p_r, b2p_r, w3p_r,
           w3a_r, b3_r, w4_r, w5_r, b45_r, out_ref):
    gv = g_ref[...]
    x3 = xyzn_ref[...]
    u = jnp.dot(x3, w1a_r[...],
                preferred_element_type=jnp.float32) + b1_r[...]
    gw = jnp.dot(gv, w1p_r[...],
                 preferred_element_type=jnp.float32)
    h3 = jnp.maximum(gw.reshape(blk, k, 32) + u[:, None, :], 0.0)
    h = h3.reshape(rows, 32)
    dk = jnp.dot(h, w2p_r[...],
                 preferred_element_type=jnp.float32) + b2p_r[...]
    dk3 = dk.reshape(blk, k, _DP)
    m = jnp.max(dk3, axis=1, keepdims=True)
    e = jnp.exp(dk3 - m)
    den = jnp.sum(e, axis=1) + 1e-08
    g3 = gv.reshape(blk, k, _DP)
    agg = jnp.sum(e * g3, axis=1) / den
    s3 = jnp.sum(e[:, :, 0:3], axis=1) / den[:, 0:3]
    t = jnp.maximum(
        jnp.dot(agg, w3p_r[...], preferred_element_type=jnp.float32)
        + jnp.dot(s3 * x3, w3a_r[...], preferred_element_type=jnp.float32)
        + b3_r[...], 0.0)
    out_ref[...] = (
        jnp.dot(t, w4_r[...], preferred_element_type=jnp.float32)
        + jnp.dot(pts_ref[...], w5_r[...], preferred_element_type=jnp.float32)
        + b45_r[...])

  full = lambda shape: pl.BlockSpec(shape, lambda i: (0, 0))
  return pl.pallas_call(
      body,
      grid=grid,
      in_specs=[
          pl.BlockSpec((rows, _DP), lambda i: (i, 0)),
          pl.BlockSpec((blk, 3), lambda i: (i, 0)),
          pl.BlockSpec((blk, c), lambda i: (i, 0)),
          full(w1p.shape), full(w1a.shape), full(b1.shape),
          full(w2p.shape), full(b2p.shape), full(w3p.shape),
          full(w3a.shape), full(b3.shape), full(w4.shape),
          full(w5.shape), full(b45.shape),
      ],
      out_specs=pl.BlockSpec((blk, w4.shape[1]), lambda i: (i, 0)),
      out_shape=jax.ShapeDtypeStruct((n, w4.shape[1]), jnp.float32),
  )(g, xyzn, pts, w1p, w1a, b1, w2p, b2p, w3p, w3a, b3, w4, w5, b45)


def _pick_block(n):
  for blk in (1000, 800, 400, 200, 80, 40, 16, 8):
    if n % blk == 0:
      return blk
  return n


def kernel(xyz, xyz_nn, points, knn, dists, W1, b1, W2, b2, W3, b3, W4, b4,
           W5, b5):
  B, N, K = knn.shape
  C = points.shape[-1]
  din = C + 4
  BN = B * N
  E = BN * K

  f32 = jnp.float32
  table = jnp.concatenate(
      [
          xyz_nn.reshape(BN, 3).astype(f32),
          jnp.zeros((BN, 1), f32),
          points.reshape(BN, C).astype(f32),
          jnp.zeros((BN, _DP - din), f32),
      ],
      axis=1)

  idx = knn.astype(jnp.int32)
  if B > 1:
    idx = idx + (jnp.arange(B, dtype=jnp.int32) * N)[:, None, None]
  idx = idx.reshape(-1)

  n_sl = 2
  while n_sl > 1 and BN % n_sl:
    n_sl -= 1
  bn_s = BN // n_sl
  e_s = bn_s * K
  per_w = -(-e_s // (_CR * _CH * _NC * _NS))
  per_w = per_w + (per_w % 2)
  e_pad = per_w * _CR * _CH * _NC * _NS
  idx_sl = jnp.pad(idx.reshape(n_sl, e_s), ((0, 0), (0, e_pad - e_s)))
  d_sl = jnp.pad(dists.astype(f32).reshape(n_sl, e_s),
                 ((0, 0), (0, e_pad - e_s)))

  xyzn = xyz.reshape(BN, 3).astype(f32)

  pad_rows = lambda w, r: jnp.pad(w.astype(f32), ((0, r - w.shape[0]), (0, 0)))
  w1p = pad_rows(W1, _DP)
  w1a = -W1[0:3].astype(f32)
  w2p = jnp.pad(W2.astype(f32), ((0, 0), (0, _DP - din)))
  b2p = jnp.pad(b2.astype(f32), (0, _DP - din)).reshape(1, _DP)
  w3p = pad_rows(W3, _DP)
  w3a = -W3[0:3].astype(f32)

  pts = points.reshape(BN, C).astype(f32)
  b1r = b1.astype(f32).reshape(1, -1)
  b3r = b3.astype(f32).reshape(1, -1)
  b45 = (b4 + b5).astype(f32).reshape(1, -1)
  w4 = W4.astype(f32)
  w5 = W5.astype(f32)
  blk = _pick_block(bn_s)

  outs = []
  for i in range(n_sl):
    g = _sc_gather(table, idx_sl[i].reshape(e_pad // _CH, _CH), d_sl[i],
                   e_pad, per_w)
    sl = slice(i * bn_s, (i + 1) * bn_s)
    outs.append(_tc_compute(
        g, xyzn[sl], pts[sl], w1p, w1a, b1r, w2p, b2p, w3p, w3a, b3r,
        w4, w5, b45, bn_s, K, C, blk))
  out = jnp.concatenate(outs, axis=0)

  return (xyz, out.reshape(B, N, -1))


# speedup vs baseline: 1.8230x; 1.8230x over previous
import functools

import jax
import jax.numpy as jnp
from jax import lax
from jax.experimental import pallas as pl
from jax.experimental.pallas import tpu as pltpu
from jax.experimental.pallas import tpu_sc as plsc

_NC = 2
_NS = 16
_CH = 128
_CR = 1
_DP = 128


def _sc_gather(table, idx2d, dists_flat, e_pad, per_w):
  assert per_w % 2 == 0
  mesh = plsc.VectorSubcoreMesh(core_axis_name="c", subcore_axis_name="s")

  @functools.partial(
      pl.kernel,
      mesh=mesh,
      compiler_params=pltpu.CompilerParams(needs_layout_passes=False),
      out_type=jax.ShapeDtypeStruct((e_pad, _DP), jnp.float32),
      scratch_types=[
          pltpu.VMEM((2, _CR, _CH), jnp.int32),
          pltpu.VMEM((2, _CR * _CH), jnp.float32),
          pltpu.VMEM((2, _CR * _CH, _DP), jnp.float32),
          pltpu.SemaphoreType.DMA, pltpu.SemaphoreType.DMA,
          pltpu.SemaphoreType.DMA, pltpu.SemaphoreType.DMA,
          pltpu.SemaphoreType.DMA, pltpu.SemaphoreType.DMA,
          pltpu.SemaphoreType.DMA, pltpu.SemaphoreType.DMA,
      ],
  )
  def gather_kernel(table_hbm, idx_hbm, dist_hbm, out_hbm, idx_v, dist_v,
                    rows_v, si0, si1, sd0, sd1, so0, so1, sg0, sg1):
    wid = lax.axis_index("s") * _NC + lax.axis_index("c")
    base = wid * per_w
    lanes = lax.iota(jnp.int32, 16)
    col3 = jnp.full((16,), 3, jnp.int32)
    si = (si0, si1)
    sd = (sd0, sd1)
    so = (so0, so1)
    sg = (sg0, sg1)

    ch = _CR * _CH

    def in_idx(row, b):
      return pltpu.make_async_copy(idx_hbm.at[pl.ds(row * _CR, _CR)],
                                   idx_v.at[b], si[b])

    def in_dist(row, b):
      return pltpu.make_async_copy(dist_hbm.at[pl.ds(row * ch, ch)],
                                   dist_v.at[b], sd[b])

    def gathers(b):
      return [
          pltpu.make_async_copy(table_hbm.at[idx_v.at[b, r]],
                                rows_v.at[b, pl.ds(r * _CH, _CH)], sg[b])
          for r in range(_CR)
      ]

    def out_copy(row, b):
      return pltpu.make_async_copy(rows_v.at[b],
                                   out_hbm.at[pl.ds(row * ch, ch)], so[b])

    in_idx(base, 0).start()
    in_dist(base, 0).start()
    in_idx(base + 1, 1).start()
    in_dist(base + 1, 1).start()
    in_idx(base, 0).wait()
    for cp in gathers(0):
      cp.start()

    def body(jj, carry):
      for b in (0, 1):
        j = jj * 2 + b
        nb = 1 - b
        row = base + j

        @pl.when(j + 1 < per_w)
        def _fire_next_gather():
          in_idx(row + 1, nb).wait()

          @pl.when(j >= 1)
          def _wait_old_out():
            out_copy(row - 1, nb).wait()

          for cp in gathers(nb):
            cp.start()

        for cp in gathers(b):
          cp.wait()
        in_dist(row, b).wait()
        for grp in range(ch // 16):
          dvec = dist_v[b, pl.ds(grp * 16, 16)]
          plsc.store_scatter(rows_v.at[b], [grp * 16 + lanes, col3], dvec)
        out_copy(row, b).start()

        @pl.when(j + 2 < per_w)
        def _start_next_in():
          in_idx(row + 2, b).start()
          in_dist(row + 2, b).start()
      return carry

    lax.fori_loop(0, per_w // 2, body, 0)
    out_copy(base + per_w - 2, 0).wait()
    out_copy(base + per_w - 1, 1).wait()

  return gather_kernel(table, idx2d, dists_flat)


def _tc_compute(g, xyzn, pts, w1p, w1a, b1, w2p, b2p, w3p, w3a, b3, w4, w5,
                b45, n, k, c, blk):
  grid = (n // blk,)
  rows = blk * k

  def body(g_ref, xyzn_ref, pts_ref, w1p_r, w1a_r, b1_r, w2p_r, b2p_r, w3p_r,
           w3a_r, b3_r, w4_r, w5_r, b45_r, out_ref):
    gv = g_ref[...]
    x3 = xyzn_ref[...]
    u = jnp.dot(x3, w1a_r[...],
                preferred_element_type=jnp.float32) + b1_r[...]
    gw = jnp.dot(gv, w1p_r[...],
                 preferred_element_type=jnp.float32)
    h3 = jnp.maximum(gw.reshape(blk, k, 32) + u[:, None, :], 0.0)
    h = h3.reshape(rows, 32)
    dk = jnp.dot(h, w2p_r[...],
                 preferred_element_type=jnp.float32) + b2p_r[...]
    dk3 = dk.reshape(blk, k, _DP)
    m = jnp.max(dk3, axis=1, keepdims=True)
    e = jnp.exp(dk3 - m)
    den = jnp.sum(e, axis=1) + 1e-08
    g3 = gv.reshape(blk, k, _DP)
    agg = jnp.sum(e * g3, axis=1) / den
    s3 = jnp.sum(e[:, :, 0:3], axis=1) / den[:, 0:3]
    t = jnp.maximum(
        jnp.dot(agg, w3p_r[...], preferred_element_type=jnp.float32)
        + jnp.dot(s3 * x3, w3a_r[...], preferred_element_type=jnp.float32)
        + b3_r[...], 0.0)
    out_ref[...] = (
        jnp.dot(t, w4_r[...], preferred_element_type=jnp.float32)
        + jnp.dot(pts_ref[...], w5_r[...], preferred_element_type=jnp.float32)
        + b45_r[...])

  full = lambda shape: pl.BlockSpec(shape, lambda i: (0, 0))
  return pl.pallas_call(
      body,
      grid=grid,
      in_specs=[
          pl.BlockSpec((rows, _DP), lambda i: (i, 0)),
          pl.BlockSpec((blk, 3), lambda i: (i, 0)),
          pl.BlockSpec((blk, c), lambda i: (i, 0)),
          full(w1p.shape), full(w1a.shape), full(b1.shape),
          full(w2p.shape), full(b2p.shape), full(w3p.shape),
          full(w3a.shape), full(b3.shape), full(w4.shape),
          full(w5.shape), full(b45.shape),
      ],
      out_specs=pl.BlockSpec((blk, w4.shape[1]), lambda i: (i, 0)),
      out_shape=jax.ShapeDtypeStruct((n, w4.shape[1]), jnp.float32),
  )(g, xyzn, pts, w1p, w1a, b1, w2p, b2p, w3p, w3a, b3, w4, w5, b45)


def _pick_block(n):
  for blk in (1000, 800, 400, 200, 80, 40, 16, 8):
    if n % blk == 0:
      return blk
  return n


def kernel(xyz, xyz_nn, points, knn, dists, W1, b1, W2, b2, W3, b3, W4, b4,
           W5, b5):
  B, N, K = knn.shape
  C = points.shape[-1]
  din = C + 4
  BN = B * N
  E = BN * K

  f32 = jnp.float32
  table = jnp.concatenate(
      [
          xyz_nn.reshape(BN, 3).astype(f32),
          jnp.zeros((BN, 1), f32),
          points.reshape(BN, C).astype(f32),
          jnp.zeros((BN, _DP - din), f32),
      ],
      axis=1)

  idx = knn.astype(jnp.int32)
  if B > 1:
    idx = idx + (jnp.arange(B, dtype=jnp.int32) * N)[:, None, None]
  idx = idx.reshape(-1)

  n_sl = 2
  while n_sl > 1 and BN % n_sl:
    n_sl -= 1
  bn_s = BN // n_sl
  e_s = bn_s * K
  per_w = -(-e_s // (_CR * _CH * _NC * _NS))
  per_w = per_w + (per_w % 2)
  e_pad = per_w * _CR * _CH * _NC * _NS
  idx_sl = jnp.pad(idx.reshape(n_sl, e_s), ((0, 0), (0, e_pad - e_s)))
  d_sl = jnp.pad(dists.astype(f32).reshape(n_sl, e_s),
                 ((0, 0), (0, e_pad - e_s)))

  xyzn = xyz.reshape(BN, 3).astype(f32)

  pad_rows = lambda w, r: jnp.pad(w.astype(f32), ((0, r - w.shape[0]), (0, 0)))
  w1p = pad_rows(W1, _DP)
  w1a = -W1[0:3].astype(f32)
  w2p = jnp.pad(W2.astype(f32), ((0, 0), (0, _DP - din)))
  b2p = jnp.pad(b2.astype(f32), (0, _DP - din)).reshape(1, _DP)
  w3p = pad_rows(W3, _DP)
  w3a = -W3[0:3].astype(f32)

  pts = points.reshape(BN, C).astype(f32)
  b1r = b1.astype(f32).reshape(1, -1)
  b3r = b3.astype(f32).reshape(1, -1)
  b45 = (b4 + b5).astype(f32).reshape(1, -1)
  w4 = W4.astype(f32)
  w5 = W5.astype(f32)
  blk = _pick_block(bn_s)

  outs = []
  for i in range(n_sl):
    g = _sc_gather(table, idx_sl[i].reshape(e_pad // _CH, _CH), d_sl[i],
                   e_pad, per_w)
    sl = slice(i * bn_s, (i + 1) * bn_s)
    outs.append(_tc_compute(
        g, xyzn[sl], pts[sl], w1p, w1a, b1r, w2p, b2p, w3p, w3a, b3r,
        w4, w5, b45, bn_s, K, C, blk))
  out = jnp.concatenate(outs, axis=0)

  return (xyz, out.reshape(B, N, -1))


# speedup vs baseline: 1.8238x; 1.0004x over previous
import functools

import jax
import jax.numpy as jnp
from jax import lax
from jax.experimental import pallas as pl
from jax.experimental.pallas import tpu as pltpu
from jax.experimental.pallas import tpu_sc as plsc

_NC = 2
_NS = 16
_CH = 128
_CR = 1
_DP = 128


def _sc_gather(table, idx2d, dists_flat, e_pad, per_w):
  assert per_w % 2 == 0
  mesh = plsc.VectorSubcoreMesh(core_axis_name="c", subcore_axis_name="s")

  @functools.partial(
      pl.kernel,
      mesh=mesh,
      compiler_params=pltpu.CompilerParams(needs_layout_passes=False),
      out_type=jax.ShapeDtypeStruct((e_pad, _DP), jnp.float32),
      scratch_types=[
          pltpu.VMEM((2, _CR, _CH), jnp.int32),
          pltpu.VMEM((2, _CR * _CH), jnp.float32),
          pltpu.VMEM((2, _CR * _CH, _DP), jnp.float32),
          pltpu.SemaphoreType.DMA, pltpu.SemaphoreType.DMA,
          pltpu.SemaphoreType.DMA, pltpu.SemaphoreType.DMA,
          pltpu.SemaphoreType.DMA, pltpu.SemaphoreType.DMA,
          pltpu.SemaphoreType.DMA, pltpu.SemaphoreType.DMA,
      ],
  )
  def gather_kernel(table_hbm, idx_hbm, dist_hbm, out_hbm, idx_v, dist_v,
                    rows_v, si0, si1, sd0, sd1, so0, so1, sg0, sg1):
    wid = lax.axis_index("s") * _NC + lax.axis_index("c")
    base = wid * per_w
    lanes = lax.iota(jnp.int32, 16)
    col3 = jnp.full((16,), 3, jnp.int32)
    si = (si0, si1)
    sd = (sd0, sd1)
    so = (so0, so1)
    sg = (sg0, sg1)

    ch = _CR * _CH

    def in_idx(row, b):
      return pltpu.make_async_copy(idx_hbm.at[pl.ds(row * _CR, _CR)],
                                   idx_v.at[b], si[b])

    def in_dist(row, b):
      return pltpu.make_async_copy(dist_hbm.at[pl.ds(row * ch, ch)],
                                   dist_v.at[b], sd[b])

    def gathers(b):
      return [
          pltpu.make_async_copy(table_hbm.at[idx_v.at[b, r]],
                                rows_v.at[b, pl.ds(r * _CH, _CH)], sg[b])
          for r in range(_CR)
      ]

    def out_copy(row, b):
      return pltpu.make_async_copy(rows_v.at[b],
                                   out_hbm.at[pl.ds(row * ch, ch)], so[b])

    in_idx(base, 0).start()
    in_dist(base, 0).start()
    in_idx(base + 1, 1).start()
    in_dist(base + 1, 1).start()
    in_idx(base, 0).wait()
    for cp in gathers(0):
      cp.start()

    def body(jj, carry):
      for b in (0, 1):
        j = jj * 2 + b
        nb = 1 - b
        row = base + j

        @pl.when(j + 1 < per_w)
        def _fire_next_gather():
          in_idx(row + 1, nb).wait()

          @pl.when(j >= 1)
          def _wait_old_out():
            out_copy(row - 1, nb).wait()

          for cp in gathers(nb):
            cp.start()

        for cp in gathers(b):
          cp.wait()
        in_dist(row, b).wait()
        for grp in range(ch // 16):
          dvec = dist_v[b, pl.ds(grp * 16, 16)]
          plsc.store_scatter(rows_v.at[b], [grp * 16 + lanes, col3], dvec)
        out_copy(row, b).start()

        @pl.when(j + 2 < per_w)
        def _start_next_in():
          in_idx(row + 2, b).start()
          in_dist(row + 2, b).start()
      return carry

    lax.fori_loop(0, per_w // 2, body, 0)
    out_copy(base + per_w - 2, 0).wait()
    out_copy(base + per_w - 1, 1).wait()

  return gather_kernel(table, idx2d, dists_flat)


def _tc_compute(g, xyzn, pts, w1p, w1a, b1, w2p, b2p, w3p, w3a, b3, w4, w5,
                b45, n, k, c, blk):
  grid = (n // blk,)
  rows = blk * k

  def body(g_ref, xyzn_ref, pts_ref, w1p_r, w1a_r, b1_r, w2p_r, b2p_r, w3p_r,
           w3a_r, b3_r, w4_r, w5_r, b45_r, out_ref):
    gv = g_ref[...]
    x3 = xyzn_ref[...]
    u = jnp.dot(x3, w1a_r[...],
                preferred_element_type=jnp.float32) + b1_r[...]
    gw = jnp.dot(gv, w1p_r[...],
                 preferred_element_type=jnp.float32)
    h3 = jnp.maximum(gw.reshape(blk, k, 32) + u[:, None, :], 0.0)
    h = h3.reshape(rows, 32)
    dk = jnp.dot(h, w2p_r[...],
                 preferred_element_type=jnp.float32) + b2p_r[...]
    dk3 = dk.reshape(blk, k, _DP)
    m = jnp.max(dk3, axis=1, keepdims=True)
    e = jnp.exp(dk3 - m)
    den = jnp.sum(e, axis=1) + 1e-08
    g3 = gv.reshape(blk, k, _DP)
    agg = jnp.sum(e * g3, axis=1) / den
    s3 = jnp.sum(e[:, :, 0:3], axis=1) / den[:, 0:3]
    t = jnp.maximum(
        jnp.dot(agg, w3p_r[...], preferred_element_type=jnp.float32)
        + jnp.dot(s3 * x3, w3a_r[...], preferred_element_type=jnp.float32)
        + b3_r[...], 0.0)
    out_ref[...] = (
        jnp.dot(t, w4_r[...], preferred_element_type=jnp.float32)
        + jnp.dot(pts_ref[...], w5_r[...], preferred_element_type=jnp.float32)
        + b45_r[...])

  full = lambda shape: pl.BlockSpec(shape, lambda i: (0, 0))
  return pl.pallas_call(
      body,
      grid=grid,
      in_specs=[
          pl.BlockSpec((rows, _DP), lambda i: (i, 0)),
          pl.BlockSpec((blk, 3), lambda i: (i, 0)),
          pl.BlockSpec((blk, c), lambda i: (i, 0)),
          full(w1p.shape), full(w1a.shape), full(b1.shape),
          full(w2p.shape), full(b2p.shape), full(w3p.shape),
          full(w3a.shape), full(b3.shape), full(w4.shape),
          full(w5.shape), full(b45.shape),
      ],
      out_specs=pl.BlockSpec((blk, w4.shape[1]), lambda i: (i, 0)),
      out_shape=jax.ShapeDtypeStruct((n, w4.shape[1]), jnp.float32),
  )(g, xyzn, pts, w1p, w1a, b1, w2p, b2p, w3p, w3a, b3, w4, w5, b45)


def _pick_block(n):
  for blk in (400, 1000, 800, 200, 80, 40, 16, 8):
    if n % blk == 0:
      return blk
  return n


def kernel(xyz, xyz_nn, points, knn, dists, W1, b1, W2, b2, W3, b3, W4, b4,
           W5, b5):
  B, N, K = knn.shape
  C = points.shape[-1]
  din = C + 4
  BN = B * N
  E = BN * K

  f32 = jnp.float32
  table = jnp.concatenate(
      [
          xyz_nn.reshape(BN, 3).astype(f32),
          jnp.zeros((BN, 1), f32),
          points.reshape(BN, C).astype(f32),
          jnp.zeros((BN, _DP - din), f32),
      ],
      axis=1)

  idx = knn.astype(jnp.int32)
  if B > 1:
    idx = idx + (jnp.arange(B, dtype=jnp.int32) * N)[:, None, None]
  idx = idx.reshape(-1)

  n_sl = 2
  while n_sl > 1 and BN % n_sl:
    n_sl -= 1
  bn_s = BN // n_sl
  e_s = bn_s * K
  per_w = -(-e_s // (_CR * _CH * _NC * _NS))
  per_w = per_w + (per_w % 2)
  e_pad = per_w * _CR * _CH * _NC * _NS
  idx_sl = jnp.pad(idx.reshape(n_sl, e_s), ((0, 0), (0, e_pad - e_s)))
  d_sl = jnp.pad(dists.astype(f32).reshape(n_sl, e_s),
                 ((0, 0), (0, e_pad - e_s)))

  xyzn = xyz.reshape(BN, 3).astype(f32)

  pad_rows = lambda w, r: jnp.pad(w.astype(f32), ((0, r - w.shape[0]), (0, 0)))
  w1p = pad_rows(W1, _DP)
  w1a = -W1[0:3].astype(f32)
  w2p = jnp.pad(W2.astype(f32), ((0, 0), (0, _DP - din)))
  b2p = jnp.pad(b2.astype(f32), (0, _DP - din)).reshape(1, _DP)
  w3p = pad_rows(W3, _DP)
  w3a = -W3[0:3].astype(f32)

  pts = points.reshape(BN, C).astype(f32)
  b1r = b1.astype(f32).reshape(1, -1)
  b3r = b3.astype(f32).reshape(1, -1)
  b45 = (b4 + b5).astype(f32).reshape(1, -1)
  w4 = W4.astype(f32)
  w5 = W5.astype(f32)
  blk = _pick_block(bn_s)

  outs = []
  for i in range(n_sl):
    g = _sc_gather(table, idx_sl[i].reshape(e_pad // _CH, _CH), d_sl[i],
                   e_pad, per_w)
    sl = slice(i * bn_s, (i + 1) * bn_s)
    outs.append(_tc_compute(
        g, xyzn[sl], pts[sl], w1p, w1a, b1r, w2p, b2p, w3p, w3a, b3r,
        w4, w5, b45, bn_s, K, C, blk))
  out = jnp.concatenate(outs, axis=0)

  return (xyz, out.reshape(B, N, -1))


# speedup vs baseline: 1.8256x; 1.0010x over previous
import functools

import jax
import jax.numpy as jnp
from jax import lax
from jax.experimental import pallas as pl
from jax.experimental.pallas import tpu as pltpu
from jax.experimental.pallas import tpu_sc as plsc

_NC = 2
_NS = 16
_CH = 128
_CR = 1
_DP = 128


def _sc_gather(table, idx2d, dists_flat, e_pad, per_w):
  assert per_w % 2 == 0
  mesh = plsc.VectorSubcoreMesh(core_axis_name="c", subcore_axis_name="s")

  @functools.partial(
      pl.kernel,
      mesh=mesh,
      compiler_params=pltpu.CompilerParams(needs_layout_passes=False),
      out_type=jax.ShapeDtypeStruct((e_pad, _DP), jnp.float32),
      scratch_types=[
          pltpu.VMEM((2, _CR, _CH), jnp.int32),
          pltpu.VMEM((2, _CR * _CH), jnp.float32),
          pltpu.VMEM((2, _CR * _CH, _DP), jnp.float32),
          pltpu.SemaphoreType.DMA, pltpu.SemaphoreType.DMA,
          pltpu.SemaphoreType.DMA, pltpu.SemaphoreType.DMA,
          pltpu.SemaphoreType.DMA, pltpu.SemaphoreType.DMA,
          pltpu.SemaphoreType.DMA, pltpu.SemaphoreType.DMA,
      ],
  )
  def gather_kernel(table_hbm, idx_hbm, dist_hbm, out_hbm, idx_v, dist_v,
                    rows_v, si0, si1, sd0, sd1, so0, so1, sg0, sg1):
    wid = lax.axis_index("s") * _NC + lax.axis_index("c")
    base = wid * per_w
    lanes = lax.iota(jnp.int32, 16)
    col3 = jnp.full((16,), 3, jnp.int32)
    si = (si0, si1)
    sd = (sd0, sd1)
    so = (so0, so1)
    sg = (sg0, sg1)

    ch = _CR * _CH

    def in_idx(row, b):
      return pltpu.make_async_copy(idx_hbm.at[pl.ds(row * _CR, _CR)],
                                   idx_v.at[b], si[b])

    def in_dist(row, b):
      return pltpu.make_async_copy(dist_hbm.at[pl.ds(row * ch, ch)],
                                   dist_v.at[b], sd[b])

    def gathers(b):
      return [
          pltpu.make_async_copy(table_hbm.at[idx_v.at[b, r]],
                                rows_v.at[b, pl.ds(r * _CH, _CH)], sg[b])
          for r in range(_CR)
      ]

    def out_copy(row, b):
      return pltpu.make_async_copy(rows_v.at[b],
                                   out_hbm.at[pl.ds(row * ch, ch)], so[b])

    in_idx(base, 0).start()
    in_dist(base, 0).start()
    in_idx(base + 1, 1).start()
    in_dist(base + 1, 1).start()
    in_idx(base, 0).wait()
    for cp in gathers(0):
      cp.start()

    def body(jj, carry):
      for b in (0, 1):
        j = jj * 2 + b
        nb = 1 - b
        row = base + j

        @pl.when(j + 1 < per_w)
        def _fire_next_gather():
          in_idx(row + 1, nb).wait()

          @pl.when(j >= 1)
          def _wait_old_out():
            out_copy(row - 1, nb).wait()

          for cp in gathers(nb):
            cp.start()

        for cp in gathers(b):
          cp.wait()
        in_dist(row, b).wait()
        for grp in range(ch // 16):
          dvec = dist_v[b, pl.ds(grp * 16, 16)]
          plsc.store_scatter(rows_v.at[b], [grp * 16 + lanes, col3], dvec)
        out_copy(row, b).start()

        @pl.when(j + 2 < per_w)
        def _start_next_in():
          in_idx(row + 2, b).start()
          in_dist(row + 2, b).start()
      return carry

    lax.fori_loop(0, per_w // 2, body, 0)
    out_copy(base + per_w - 2, 0).wait()
    out_copy(base + per_w - 1, 1).wait()

  return gather_kernel(table, idx2d, dists_flat)


def _tc_compute(g, xyzn, pts, w1p, w1a, b1, w2p, b2p, w3p, w3a, b3, w4, w5,
                b45, n, k, c, blk):
  grid = (n // blk,)
  rows = blk * k

  def body(g_ref, xyzn_ref, pts_ref, w1p_r, w1a_r, b1_r, w2p_r, b2p_r, w3p_r,
           w3a_r, b3_r, w4_r, w5_r, b45_r, out_ref):
    gv = g_ref[...]
    x3 = xyzn_ref[...]
    u = jnp.dot(x3, w1a_r[...],
                preferred_element_type=jnp.float32) + b1_r[...]
    gw = jnp.dot(gv, w1p_r[...],
                 preferred_element_type=jnp.float32)
    h3 = jnp.maximum(gw.reshape(blk, k, 32) + u[:, None, :], 0.0)
    h = h3.reshape(rows, 32)
    dk = jnp.dot(h, w2p_r[...],
                 preferred_element_type=jnp.float32) + b2p_r[...]
    dk3 = dk.reshape(blk, k, _DP)
    m = jnp.max(dk3, axis=1, keepdims=True)
    e = jnp.exp(dk3 - m)
    s = jnp.sum(e, axis=1, keepdims=True) + 1e-08
    sm = e / s
    g3 = gv.reshape(blk, k, _DP)
    agg = jnp.sum(sm * g3, axis=1)
    s3 = jnp.sum(sm[:, :, 0:3], axis=1)
    t = jnp.maximum(
        jnp.dot(agg, w3p_r[...], preferred_element_type=jnp.float32)
        + jnp.dot(s3 * x3, w3a_r[...], preferred_element_type=jnp.float32)
        + b3_r[...], 0.0)
    out_ref[...] = (
        jnp.dot(t, w4_r[...], preferred_element_type=jnp.float32)
        + jnp.dot(pts_ref[...], w5_r[...], preferred_element_type=jnp.float32)
        + b45_r[...])

  full = lambda shape: pl.BlockSpec(shape, lambda i: (0, 0))
  return pl.pallas_call(
      body,
      grid=grid,
      in_specs=[
          pl.BlockSpec((rows, _DP), lambda i: (i, 0)),
          pl.BlockSpec((blk, 3), lambda i: (i, 0)),
          pl.BlockSpec((blk, c), lambda i: (i, 0)),
          full(w1p.shape), full(w1a.shape), full(b1.shape),
          full(w2p.shape), full(b2p.shape), full(w3p.shape),
          full(w3a.shape), full(b3.shape), full(w4.shape),
          full(w5.shape), full(b45.shape),
      ],
      out_specs=pl.BlockSpec((blk, w4.shape[1]), lambda i: (i, 0)),
      out_shape=jax.ShapeDtypeStruct((n, w4.shape[1]), jnp.float32),
  )(g, xyzn, pts, w1p, w1a, b1, w2p, b2p, w3p, w3a, b3, w4, w5, b45)


def _pick_block(n):
  for blk in (400, 1000, 800, 200, 80, 40, 16, 8):
    if n % blk == 0:
      return blk
  return n


def kernel(xyz, xyz_nn, points, knn, dists, W1, b1, W2, b2, W3, b3, W4, b4,
           W5, b5):
  B, N, K = knn.shape
  C = points.shape[-1]
  din = C + 4
  BN = B * N
  E = BN * K

  f32 = jnp.float32
  table = jnp.concatenate(
      [
          xyz_nn.reshape(BN, 3).astype(f32),
          jnp.zeros((BN, 1), f32),
          points.reshape(BN, C).astype(f32),
          jnp.zeros((BN, _DP - din), f32),
      ],
      axis=1)

  idx = knn.astype(jnp.int32)
  if B > 1:
    idx = idx + (jnp.arange(B, dtype=jnp.int32) * N)[:, None, None]
  idx = idx.reshape(-1)

  n_sl = 2
  while n_sl > 1 and BN % n_sl:
    n_sl -= 1
  bn_s = BN // n_sl
  e_s = bn_s * K
  per_w = -(-e_s // (_CR * _CH * _NC * _NS))
  per_w = per_w + (per_w % 2)
  e_pad = per_w * _CR * _CH * _NC * _NS
  idx_sl = jnp.pad(idx.reshape(n_sl, e_s), ((0, 0), (0, e_pad - e_s)))
  d_sl = jnp.pad(dists.astype(f32).reshape(n_sl, e_s),
                 ((0, 0), (0, e_pad - e_s)))

  xyzn = xyz.reshape(BN, 3).astype(f32)

  pad_rows = lambda w, r: jnp.pad(w.astype(f32), ((0, r - w.shape[0]), (0, 0)))
  w1p = pad_rows(W1, _DP)
  w1a = -W1[0:3].astype(f32)
  w2p = jnp.pad(W2.astype(f32), ((0, 0), (0, _DP - din)))
  b2p = jnp.pad(b2.astype(f32), (0, _DP - din)).reshape(1, _DP)
  w3p = pad_rows(W3, _DP)
  w3a = -W3[0:3].astype(f32)

  pts = points.reshape(BN, C).astype(f32)
  b1r = b1.astype(f32).reshape(1, -1)
  b3r = b3.astype(f32).reshape(1, -1)
  b45 = (b4 + b5).astype(f32).reshape(1, -1)
  w4 = W4.astype(f32)
  w5 = W5.astype(f32)
  blk = _pick_block(bn_s)

  outs = []
  for i in range(n_sl):
    g = _sc_gather(table, idx_sl[i].reshape(e_pad // _CH, _CH), d_sl[i],
                   e_pad, per_w)
    sl = slice(i * bn_s, (i + 1) * bn_s)
    outs.append(_tc_compute(
        g, xyzn[sl], pts[sl], w1p, w1a, b1r, w2p, b2p, w3p, w3a, b3r,
        w4, w5, b45, bn_s, K, C, blk))
  out = jnp.concatenate(outs, axis=0)

  return (xyz, out.reshape(B, N, -1))
